# baseline (speedup 1.0000x reference)
import jax
import jax.numpy as jnp
from jax import lax
from jax.experimental import pallas as pl
from jax.experimental.pallas import tpu as pltpu
from jax.experimental.pallas import tpu_sc as plsc

N = 10000
E = 320000
D = 128
D_EDGE = 16
STEPS = 3

NC, NS = 2, 16
NW = NC * NS
EW = E // NW
CH = 80
NCHUNK = EW // CH
NPAD = 10240
RPT = NPAD // NS
RCH = 128
NRCH = RPT // RCH

_MESH = plsc.VectorSubcoreMesh(core_axis_name="c", subcore_axis_name="s")


GRP = 8
NGRP = NCHUNK // GRP
GTAIL = NCHUNK - NGRP * GRP


def _gather_body(table, idx_hbm, out, *scratch):
    idx_v = scratch[0:GRP]
    rows_v = scratch[GRP:2 * GRP]
    isem, gsem, wsem = scratch[2 * GRP:2 * GRP + 3]
    c = lax.axis_index("c")
    s = lax.axis_index("s")
    wid = s * NC + c
    base = wid * EW

    def group(off0, n):
        idesc = [
            pltpu.async_copy(idx_hbm.at[pl.ds(off0 + b * CH, CH)], idx_v[b], isem)
            for b in range(n)
        ]
        gdesc = []
        for b in range(n):
            idesc[b].wait()
            gdesc.append(pltpu.async_copy(table.at[idx_v[b]], rows_v[b], gsem))
        wdesc = []
        for b in range(n):
            gdesc[b].wait()
            wdesc.append(
                pltpu.async_copy(rows_v[b], out.at[pl.ds(off0 + b * CH, CH)], wsem)
            )
        for b in range(n):
            wdesc[b].wait()

    def body(i, carry):
        group(base + i * (GRP * CH), GRP)
        return carry

    lax.fori_loop(0, NGRP, body, 0)
    if GTAIL:
        group(base + NGRP * GRP * CH, GTAIL)


def _sc_gather(table, idx):
    width = table.shape[1]
    dtype = table.dtype
    return pl.kernel(
        _gather_body,
        out_type=jax.ShapeDtypeStruct((E, width), dtype),
        mesh=_MESH,
        scratch_types=(
            [pltpu.VMEM((CH,), jnp.int32) for _ in range(GRP)]
            + [pltpu.VMEM((CH, width), dtype) for _ in range(GRP)]
            + [pltpu.SemaphoreType.DMA] * 3
        ),
    )(table, idx)


CHS = 40
NCHUNKS = EW // CHS
SGRP = 8
SNGRP = NCHUNKS // SGRP
STAIL = NCHUNKS - SNGRP * SGRP
ZCP = RPT // CHS


def _scatter_body(vals, idx_hbm, out, acc, *scratch):
    idx_v = scratch[0:SGRP]
    rows_v = scratch[SGRP:2 * SGRP]
    isem, vsem, ssem, zsem = scratch[2 * SGRP:2 * SGRP + 4]
    c = lax.axis_index("c")
    s = lax.axis_index("s")
    wid = s * NC + c

    def zrow(r, carry):
        for k in range(D // 16):
            rows_v[0][r, pl.ds(k * 16, 16)] = jnp.zeros((16,), jnp.float32)
        return carry

    lax.fori_loop(0, CHS, zrow, 0)

    r0 = s * RPT
    zdesc = [
        pltpu.async_copy(rows_v[0], acc.at[pl.ds(r0 + k * CHS, CHS)], zsem)
        for k in range(ZCP)
    ]
    for d in zdesc:
        d.wait()
    plsc.subcore_barrier()

    base = wid * EW

    def body(i, carry):
        off0 = base + i * (SGRP * CHS)
        idesc = [
            pltpu.async_copy(idx_hbm.at[pl.ds(off0 + b * CHS, CHS)], idx_v[b], isem)
            for b in range(SGRP)
        ]
        vdesc = [
            pltpu.async_copy(vals.at[pl.ds(off0 + b * CHS, CHS)], rows_v[b], vsem)
            for b in range(SGRP)
        ]
        sdesc = []
        for b in range(SGRP):
            idesc[b].wait()
            vdesc[b].wait()
            sdesc.append(
                pltpu.async_copy(rows_v[b], acc.at[idx_v[b]], ssem, add=True)
            )
        for b in range(SGRP):
            sdesc[b].wait()
        return carry

    lax.fori_loop(0, SNGRP, body, 0)

    for t in range(STAIL):
        off = base + (SNGRP * SGRP + t) * CHS
        pltpu.sync_copy(idx_hbm.at[pl.ds(off, CHS)], idx_v[0])
        pltpu.sync_copy(vals.at[pl.ds(off, CHS)], rows_v[0])
        pltpu.sync_copy(rows_v[0], acc.at[idx_v[0]], add=True)

    plsc.subcore_barrier()

    wdesc = [
        pltpu.async_copy(
            acc.at[pl.ds(r0 + k * RCH, RCH)],
            out.at[pl.ds(c * NPAD + r0 + k * RCH, RCH)],
            zsem,
        )
        for k in range(NRCH)
    ]
    for d in wdesc:
        d.wait()


def _sc_scatter(vals, idx):
    return pl.kernel(
        _scatter_body,
        out_type=jax.ShapeDtypeStruct((2 * NPAD, D), jnp.float32),
        mesh=_MESH,
        scratch_types=(
            [pltpu.VMEM_SHARED((NPAD, D), jnp.float32)]
            + [pltpu.VMEM((CHS,), jnp.int32) for _ in range(SGRP)]
            + [pltpu.VMEM((CHS, D), jnp.float32) for _ in range(SGRP)]
            + [pltpu.SemaphoreType.DMA] * 4
        ),
    )(vals, idx)


NB = 1000
NGRID = N // NB
EB = 8000
EGRID = E // EB
AB = 1024
AGRID = NPAD // AB


def _xw_body(x_ref, w_ref, b_ref, o_ref):
    o_ref[...] = (
        jnp.dot(x_ref[...], w_ref[...], preferred_element_type=jnp.float32)
        + b_ref[...]
    )


def _tc_node_xw(x, w, b):
    return pl.pallas_call(
        _xw_body,
        grid=(NGRID,),
        in_specs=[
            pl.BlockSpec((NB, D), lambda i: (i, 0)),
            pl.BlockSpec((D, D), lambda i: (0, 0)),
            pl.BlockSpec((1, D), lambda i: (0, 0)),
        ],
        out_specs=pl.BlockSpec((NB, D), lambda i: (i, 0)),
        out_shape=jax.ShapeDtypeStruct((N, D), jnp.float32),
    )(x, w, b)


def _h0_body(gx_ref, ef_ref, w_ref, oh_ref, o16_ref):
    h = jnp.maximum(
        gx_ref[...]
        + jnp.dot(ef_ref[...], w_ref[...], preferred_element_type=jnp.float32),
        0.0,
    )
    oh_ref[...] = h
    o16_ref[...] = h.astype(jnp.bfloat16)


def _tc_edge_h0(gx, ef, w):
    return pl.pallas_call(
        _h0_body,
        grid=(EGRID,),
        in_specs=[
            pl.BlockSpec((EB, D), lambda i: (i, 0)),
            pl.BlockSpec((EB, D_EDGE), lambda i: (i, 0)),
            pl.BlockSpec((D_EDGE, D), lambda i: (0, 0)),
        ],
        out_specs=[
            pl.BlockSpec((EB, D), lambda i: (i, 0)),
            pl.BlockSpec((EB, D), lambda i: (i, 0)),
        ],
        out_shape=[
            jax.ShapeDtypeStruct((E, D), jnp.float32),
            jax.ShapeDtypeStruct((E, D), jnp.bfloat16),
        ],
    )(gx, ef, w)


def _a_body(p0_ref, p1_ref, w_ref, b_ref, o_ref):
    o_ref[...] = (
        jnp.dot(
            p0_ref[...] + p1_ref[...], w_ref[...],
            preferred_element_type=jnp.float32,
        )
        + b_ref[...]
    )


def _tc_node_a(p, w, b):
    return pl.pallas_call(
        _a_body,
        grid=(AGRID,),
        in_specs=[
            pl.BlockSpec((AB, D), lambda i: (i, 0)),
            pl.BlockSpec((AB, D), lambda i: (i + AGRID, 0)),
            pl.BlockSpec((D, D), lambda i: (0, 0)),
            pl.BlockSpec((1, D), lambda i: (0, 0)),
        ],
        out_specs=pl.BlockSpec((AB, D), lambda i: (i, 0)),
        out_shape=jax.ShapeDtypeStruct((NPAD, D), jnp.float32),
    )(p, p, w, b)


def _ph_body(h_ref, w1_ref, o_ref):
    o_ref[...] = jnp.dot(h_ref[...], w1_ref[...],
                         preferred_element_type=jnp.float32)


def _tc_edge_ph(h, w1):
    return pl.pallas_call(
        _ph_body,
        grid=(EGRID,),
        in_specs=[
            pl.BlockSpec((EB, D), lambda i: (i, 0)),
            pl.BlockSpec((D, D), lambda i: (0, 0)),
        ],
        out_specs=pl.BlockSpec((EB, D), lambda i: (i, 0)),
        out_shape=jax.ShapeDtypeStruct((E, D), jnp.float32),
    )(h, w1)


def _step_body(g_ref, ph_ref, h0_ref, w2_ref, b2_ref, o_ref):
    t = jnp.maximum(g_ref[...] - ph_ref[...], 0.0)
    o_ref[...] = jnp.maximum(
        h0_ref[...].astype(jnp.float32)
        + jnp.dot(t, w2_ref[...], preferred_element_type=jnp.float32)
        + b2_ref[...],
        0.0,
    )


def _tc_edge_step(g, ph, h0, w2, b2):
    return pl.pallas_call(
        _step_body,
        grid=(EGRID,),
        in_specs=[
            pl.BlockSpec((EB, D), lambda i: (i, 0)),
            pl.BlockSpec((EB, D), lambda i: (i, 0)),
            pl.BlockSpec((EB, D), lambda i: (i, 0)),
            pl.BlockSpec((D, D), lambda i: (0, 0)),
            pl.BlockSpec((1, D), lambda i: (0, 0)),
        ],
        out_specs=pl.BlockSpec((EB, D), lambda i: (i, 0)),
        out_shape=jax.ShapeDtypeStruct((E, D), jnp.float32),
    )(g, ph, h0, w2, b2)


def _final_body(x_ref, p0_ref, p1_ref, wx_ref, wm_ref, b_ref, g_ref, be_ref, o_ref):
    hf = jnp.maximum(
        jnp.dot(x_ref[...], wx_ref[...], preferred_element_type=jnp.float32)
        + jnp.dot(
            p0_ref[...] + p1_ref[...], wm_ref[...],
            preferred_element_type=jnp.float32,
        )
        + b_ref[...],
        0.0,
    )
    mu = jnp.mean(hf, axis=1, keepdims=True)
    d = hf - mu
    var = jnp.mean(d * d, axis=1, keepdims=True)
    hn = d * lax.rsqrt(var + 1e-5) * g_ref[...] + be_ref[...]
    o_ref[...] = jnp.maximum(hn, 0.0)


def _tc_node_final(x, p0, p1, wx, wm, b, gam, bet):
    return pl.pallas_call(
        _final_body,
        grid=(NGRID,),
        in_specs=[
            pl.BlockSpec((NB, D), lambda i: (i, 0)),
            pl.BlockSpec((NB, D), lambda i: (i, 0)),
            pl.BlockSpec((NB, D), lambda i: (i, 0)),
            pl.BlockSpec((D, D), lambda i: (0, 0)),
            pl.BlockSpec((D, D), lambda i: (0, 0)),
            pl.BlockSpec((1, D), lambda i: (0, 0)),
            pl.BlockSpec((1, D), lambda i: (0, 0)),
            pl.BlockSpec((1, D), lambda i: (0, 0)),
        ],
        out_specs=pl.BlockSpec((NB, D), lambda i: (i, 0)),
        out_shape=jax.ShapeDtypeStruct((N, D), jnp.float32),
    )(x, p0, p1, wx, wm, b, gam, bet)


def kernel(x, edge_feats, edge_index, W_init, b_init, W_h1, b_h1, W_h2, b_h2,
           W_final, b_final, ln_gamma, ln_beta):
    src = edge_index[0]
    dst = edge_index[1]
    Wi1 = W_init[:D]
    Wi2 = W_init[D:]
    Wf1 = W_final[:D]
    Wf2 = W_final[D:]
    bi = b_init.reshape(1, D)
    b1 = b_h1.reshape(1, D)
    b2 = b_h2.reshape(1, D)
    bf = b_final.reshape(1, D)
    gam = ln_gamma.reshape(1, D)
    bet = ln_beta.reshape(1, D)

    XW = _tc_node_xw(x, Wi1, bi)
    GX = _sc_gather(XW, src)
    h, h0 = _tc_edge_h0(GX, edge_feats, Wi2)
    for _ in range(STEPS):
        P = _sc_scatter(h, dst)
        Ph = _tc_edge_ph(h, W_h1)
        A = _tc_node_a(P, W_h1, b1)
        G = _sc_gather(A, dst)
        h = _tc_edge_step(G, Ph, h0, W_h2, b2)
    P = _sc_scatter(h, src)
    return _tc_node_final(x, P[:N], P[NPAD:NPAD + N], Wf1, Wf2, bf, gam, bet)


# speedup vs baseline: 1.1715x; 1.1715x over previous
import jax
import jax.numpy as jnp
from jax import lax
from jax.experimental import pallas as pl
from jax.experimental.pallas import tpu as pltpu
from jax.experimental.pallas import tpu_sc as plsc

N = 10000
E = 320000
D = 128
D_EDGE = 16
STEPS = 3

NC, NS = 2, 16
NW = NC * NS
EW = E // NW
CH = 80
NCHUNK = EW // CH
NPAD = 10240
RPT = NPAD // NS
RCH = 128
NRCH = RPT // RCH

_MESH = plsc.VectorSubcoreMesh(core_axis_name="c", subcore_axis_name="s")


GRP = 8
NGRP = NCHUNK // GRP
GTAIL = NCHUNK - NGRP * GRP


def _gather_body(table, idx_hbm, out, *scratch):
    idx_v = scratch[0:GRP]
    rows_v = scratch[GRP:2 * GRP]
    isem, gsem, wsem = scratch[2 * GRP:2 * GRP + 3]
    c = lax.axis_index("c")
    s = lax.axis_index("s")
    wid = s * NC + c
    base = wid * EW

    def group(off0, n):
        idesc = [
            pltpu.async_copy(idx_hbm.at[pl.ds(off0 + b * CH, CH)], idx_v[b], isem)
            for b in range(n)
        ]
        gdesc = []
        for b in range(n):
            idesc[b].wait()
            gdesc.append(pltpu.async_copy(table.at[idx_v[b]], rows_v[b], gsem))
        wdesc = []
        for b in range(n):
            gdesc[b].wait()
            wdesc.append(
                pltpu.async_copy(rows_v[b], out.at[pl.ds(off0 + b * CH, CH)], wsem)
            )
        for b in range(n):
            wdesc[b].wait()

    def body(i, carry):
        group(base + i * (GRP * CH), GRP)
        return carry

    lax.fori_loop(0, NGRP, body, 0)
    if GTAIL:
        group(base + NGRP * GRP * CH, GTAIL)


def _sc_gather(table, idx):
    width = table.shape[1]
    dtype = table.dtype
    return pl.kernel(
        _gather_body,
        out_type=jax.ShapeDtypeStruct((E, width), dtype),
        mesh=_MESH,
        scratch_types=(
            [pltpu.VMEM((CH,), jnp.int32) for _ in range(GRP)]
            + [pltpu.VMEM((CH, width), dtype) for _ in range(GRP)]
            + [pltpu.SemaphoreType.DMA] * 3
        ),
    )(table, idx)


CHS = 40
NCHUNKS = EW // CHS
SGRP = 8
SNGRP = NCHUNKS // SGRP
STAIL = NCHUNKS - SNGRP * SGRP
ZCP = RPT // CHS


def _scatter_body(vals, idx_hbm, out, acc, *scratch):
    idx_v = scratch[0:SGRP]
    rows_v = scratch[SGRP:2 * SGRP]
    isem, vsem, ssem, zsem = scratch[2 * SGRP:2 * SGRP + 4]
    c = lax.axis_index("c")
    s = lax.axis_index("s")
    wid = s * NC + c

    def zrow(r, carry):
        for k in range(D // 16):
            rows_v[0][r, pl.ds(k * 16, 16)] = jnp.zeros((16,), jnp.float32)
        return carry

    lax.fori_loop(0, CHS, zrow, 0)

    r0 = s * RPT
    zdesc = [
        pltpu.async_copy(rows_v[0], acc.at[pl.ds(r0 + k * CHS, CHS)], zsem)
        for k in range(ZCP)
    ]
    for d in zdesc:
        d.wait()
    plsc.subcore_barrier()

    base = wid * EW

    def body(i, carry):
        off0 = base + i * (SGRP * CHS)
        idesc = [
            pltpu.async_copy(idx_hbm.at[pl.ds(off0 + b * CHS, CHS)], idx_v[b], isem)
            for b in range(SGRP)
        ]
        vdesc = [
            pltpu.async_copy(vals.at[pl.ds(off0 + b * CHS, CHS)], rows_v[b], vsem)
            for b in range(SGRP)
        ]
        sdesc = []
        for b in range(SGRP):
            idesc[b].wait()
            vdesc[b].wait()
            sdesc.append(
                pltpu.async_copy(rows_v[b], acc.at[idx_v[b]], ssem, add=True)
            )
        for b in range(SGRP):
            sdesc[b].wait()
        return carry

    lax.fori_loop(0, SNGRP, body, 0)

    for t in range(STAIL):
        off = base + (SNGRP * SGRP + t) * CHS
        pltpu.sync_copy(idx_hbm.at[pl.ds(off, CHS)], idx_v[0])
        pltpu.sync_copy(vals.at[pl.ds(off, CHS)], rows_v[0])
        pltpu.sync_copy(rows_v[0], acc.at[idx_v[0]], add=True)

    plsc.subcore_barrier()

    wdesc = [
        pltpu.async_copy(
            acc.at[pl.ds(r0 + k * RCH, RCH)],
            out.at[pl.ds(c * NPAD + r0 + k * RCH, RCH)],
            zsem,
        )
        for k in range(NRCH)
    ]
    for d in wdesc:
        d.wait()


def _sc_scatter(vals, idx):
    return pl.kernel(
        _scatter_body,
        out_type=jax.ShapeDtypeStruct((2 * NPAD, D), jnp.float32),
        mesh=_MESH,
        scratch_types=(
            [pltpu.VMEM_SHARED((NPAD, D), jnp.float32)]
            + [pltpu.VMEM((CHS,), jnp.int32) for _ in range(SGRP)]
            + [pltpu.VMEM((CHS, D), jnp.float32) for _ in range(SGRP)]
            + [pltpu.SemaphoreType.DMA] * 4
        ),
    )(vals, idx)


NB = 1000
NGRID = N // NB
EB = 10000
EGRID = E // EB
AB = 1024
AGRID = NPAD // AB


def _xw_body(x_ref, w_ref, b_ref, o_ref):
    o_ref[...] = (
        jnp.dot(x_ref[...], w_ref[...], preferred_element_type=jnp.float32)
        + b_ref[...]
    )


def _tc_node_xw(x, w, b):
    return pl.pallas_call(
        _xw_body,
        grid=(NGRID,),
        in_specs=[
            pl.BlockSpec((NB, D), lambda i: (i, 0)),
            pl.BlockSpec((D, D), lambda i: (0, 0)),
            pl.BlockSpec((1, D), lambda i: (0, 0)),
        ],
        out_specs=pl.BlockSpec((NB, D), lambda i: (i, 0)),
        out_shape=jax.ShapeDtypeStruct((N, D), jnp.float32),
    )(x, w, b)


def _h0_body(gx_ref, ef_ref, w_ref, oh_ref, o16_ref):
    h = jnp.maximum(
        gx_ref[...]
        + jnp.dot(ef_ref[...], w_ref[...], preferred_element_type=jnp.float32),
        0.0,
    )
    oh_ref[...] = h
    o16_ref[...] = h.astype(jnp.bfloat16)


def _tc_edge_h0(gx, ef, w):
    return pl.pallas_call(
        _h0_body,
        grid=(EGRID,),
        in_specs=[
            pl.BlockSpec((EB, D), lambda i: (i, 0)),
            pl.BlockSpec((EB, D_EDGE), lambda i: (i, 0)),
            pl.BlockSpec((D_EDGE, D), lambda i: (0, 0)),
        ],
        out_specs=[
            pl.BlockSpec((EB, D), lambda i: (i, 0)),
            pl.BlockSpec((EB, D), lambda i: (i, 0)),
        ],
        out_shape=[
            jax.ShapeDtypeStruct((E, D), jnp.float32),
            jax.ShapeDtypeStruct((E, D), jnp.bfloat16),
        ],
    )(gx, ef, w)


def _a_body(p0_ref, p1_ref, w_ref, b_ref, o_ref):
    o_ref[...] = (
        jnp.dot(
            p0_ref[...] + p1_ref[...], w_ref[...],
            preferred_element_type=jnp.float32,
        )
        + b_ref[...]
    )


def _tc_node_a(p, w, b):
    return pl.pallas_call(
        _a_body,
        grid=(AGRID,),
        in_specs=[
            pl.BlockSpec((AB, D), lambda i: (i, 0)),
            pl.BlockSpec((AB, D), lambda i: (i + AGRID, 0)),
            pl.BlockSpec((D, D), lambda i: (0, 0)),
            pl.BlockSpec((1, D), lambda i: (0, 0)),
        ],
        out_specs=pl.BlockSpec((AB, D), lambda i: (i, 0)),
        out_shape=jax.ShapeDtypeStruct((NPAD, D), jnp.float32),
    )(p, p, w, b)


def _step_body(g_ref, h_ref, h0_ref, w1_ref, w2_ref, b2_ref, o_ref):
    t = jnp.maximum(
        g_ref[...]
        - jnp.dot(h_ref[...], w1_ref[...], preferred_element_type=jnp.float32),
        0.0,
    )
    o_ref[...] = jnp.maximum(
        h0_ref[...].astype(jnp.float32)
        + jnp.dot(t, w2_ref[...], preferred_element_type=jnp.float32)
        + b2_ref[...],
        0.0,
    )


def _tc_edge_step(g, h, h0, w1, w2, b2):
    return pl.pallas_call(
        _step_body,
        grid=(EGRID,),
        in_specs=[
            pl.BlockSpec((EB, D), lambda i: (i, 0)),
            pl.BlockSpec((EB, D), lambda i: (i, 0)),
            pl.BlockSpec((EB, D), lambda i: (i, 0)),
            pl.BlockSpec((D, D), lambda i: (0, 0)),
            pl.BlockSpec((D, D), lambda i: (0, 0)),
            pl.BlockSpec((1, D), lambda i: (0, 0)),
        ],
        out_specs=pl.BlockSpec((EB, D), lambda i: (i, 0)),
        out_shape=jax.ShapeDtypeStruct((E, D), jnp.float32),
    )(g, h, h0, w1, w2, b2)


def _final_body(x_ref, p0_ref, p1_ref, wx_ref, wm_ref, b_ref, g_ref, be_ref, o_ref):
    hf = jnp.maximum(
        jnp.dot(x_ref[...], wx_ref[...], preferred_element_type=jnp.float32)
        + jnp.dot(
            p0_ref[...] + p1_ref[...], wm_ref[...],
            preferred_element_type=jnp.float32,
        )
        + b_ref[...],
        0.0,
    )
    mu = jnp.mean(hf, axis=1, keepdims=True)
    d = hf - mu
    var = jnp.mean(d * d, axis=1, keepdims=True)
    hn = d * lax.rsqrt(var + 1e-5) * g_ref[...] + be_ref[...]
    o_ref[...] = jnp.maximum(hn, 0.0)


def _tc_node_final(x, p0, p1, wx, wm, b, gam, bet):
    return pl.pallas_call(
        _final_body,
        grid=(NGRID,),
        in_specs=[
            pl.BlockSpec((NB, D), lambda i: (i, 0)),
            pl.BlockSpec((NB, D), lambda i: (i, 0)),
            pl.BlockSpec((NB, D), lambda i: (i, 0)),
            pl.BlockSpec((D, D), lambda i: (0, 0)),
            pl.BlockSpec((D, D), lambda i: (0, 0)),
            pl.BlockSpec((1, D), lambda i: (0, 0)),
            pl.BlockSpec((1, D), lambda i: (0, 0)),
            pl.BlockSpec((1, D), lambda i: (0, 0)),
        ],
        out_specs=pl.BlockSpec((NB, D), lambda i: (i, 0)),
        out_shape=jax.ShapeDtypeStruct((N, D), jnp.float32),
    )(x, p0, p1, wx, wm, b, gam, bet)


def kernel(x, edge_feats, edge_index, W_init, b_init, W_h1, b_h1, W_h2, b_h2,
           W_final, b_final, ln_gamma, ln_beta):
    src = edge_index[0]
    dst = edge_index[1]
    Wi1 = W_init[:D]
    Wi2 = W_init[D:]
    Wf1 = W_final[:D]
    Wf2 = W_final[D:]
    bi = b_init.reshape(1, D)
    b1 = b_h1.reshape(1, D)
    b2 = b_h2.reshape(1, D)
    bf = b_final.reshape(1, D)
    gam = ln_gamma.reshape(1, D)
    bet = ln_beta.reshape(1, D)

    XW = _tc_node_xw(x, Wi1, bi)
    GX = _sc_gather(XW, src)
    h, h0 = _tc_edge_h0(GX, edge_feats, Wi2)
    for _ in range(STEPS):
        P = _sc_scatter(h, dst)
        A = _tc_node_a(P, W_h1, b1)
        G = _sc_gather(A, dst)
        h = _tc_edge_step(G, h, h0, W_h1, W_h2, b2)
    P = _sc_scatter(h, src)
    return _tc_node_final(x, P[:N], P[NPAD:NPAD + N], Wf1, Wf2, bf, gam, bet)


# speedup vs baseline: 1.2196x; 1.0410x over previous
import jax
import jax.numpy as jnp
from jax import lax
from jax.experimental import pallas as pl
from jax.experimental.pallas import tpu as pltpu
from jax.experimental.pallas import tpu_sc as plsc

N = 10000
E = 320000
D = 128
D_EDGE = 16
STEPS = 3

NC, NS = 2, 16
NW = NC * NS
EW = E // NW
CH = 80
NCHUNK = EW // CH
NPAD = 10240
RPT = NPAD // NS
RCH = 128
NRCH = RPT // RCH

_MESH = plsc.VectorSubcoreMesh(core_axis_name="c", subcore_axis_name="s")


GRP = 8
NGRP = NCHUNK // GRP
GTAIL = NCHUNK - NGRP * GRP


def _gather_body(table, idx_hbm, out, *scratch):
    idx_v = scratch[0:GRP]
    rows_v = scratch[GRP:2 * GRP]
    isem, gsem, wsem = scratch[2 * GRP:2 * GRP + 3]
    c = lax.axis_index("c")
    s = lax.axis_index("s")
    wid = s * NC + c
    base = wid * EW

    def group(off0, n):
        idesc = [
            pltpu.async_copy(idx_hbm.at[pl.ds(off0 + b * CH, CH)], idx_v[b], isem)
            for b in range(n)
        ]
        gdesc = []
        for b in range(n):
            idesc[b].wait()
            gdesc.append(pltpu.async_copy(table.at[idx_v[b]], rows_v[b], gsem))
        wdesc = []
        for b in range(n):
            gdesc[b].wait()
            wdesc.append(
                pltpu.async_copy(rows_v[b], out.at[pl.ds(off0 + b * CH, CH)], wsem)
            )
        for b in range(n):
            wdesc[b].wait()

    def body(i, carry):
        group(base + i * (GRP * CH), GRP)
        return carry

    lax.fori_loop(0, NGRP, body, 0)
    if GTAIL:
        group(base + NGRP * GRP * CH, GTAIL)


def _sc_gather(table, idx):
    width = table.shape[1]
    dtype = table.dtype
    return pl.kernel(
        _gather_body,
        out_type=jax.ShapeDtypeStruct((E, width), dtype),
        mesh=_MESH,
        scratch_types=(
            [pltpu.VMEM((CH,), jnp.int32) for _ in range(GRP)]
            + [pltpu.VMEM((CH, width), dtype) for _ in range(GRP)]
            + [pltpu.SemaphoreType.DMA] * 3
        ),
    )(table, idx)


GRP2 = 4


def _gather_spmem_body(table, idx_hbm, out, tbl, *scratch):
    idx_v = scratch[0:GRP2]
    rows_v = scratch[GRP2:2 * GRP2]
    isem, gsem, wsem, lsem = scratch[2 * GRP2:2 * GRP2 + 4]
    c = lax.axis_index("c")
    s = lax.axis_index("s")
    wid = s * NC + c
    base = wid * EW

    r0 = s * RPT
    ldesc = [
        pltpu.async_copy(
            table.at[pl.ds(r0 + k * RCH, RCH)], tbl.at[pl.ds(r0 + k * RCH, RCH)],
            lsem,
        )
        for k in range(NRCH)
    ]
    for d in ldesc:
        d.wait()
    plsc.subcore_barrier()

    def group(off0, n):
        idesc = [
            pltpu.async_copy(idx_hbm.at[pl.ds(off0 + b * CH, CH)], idx_v[b], isem)
            for b in range(n)
        ]
        gdesc = []
        for b in range(n):
            idesc[b].wait()
            gdesc.append(pltpu.async_copy(tbl.at[idx_v[b]], rows_v[b], gsem))
        wdesc = []
        for b in range(n):
            gdesc[b].wait()
            wdesc.append(
                pltpu.async_copy(rows_v[b], out.at[pl.ds(off0 + b * CH, CH)], wsem)
            )
        for b in range(n):
            wdesc[b].wait()

    ngrp2 = NCHUNK // GRP2
    def body(i, carry):
        group(base + i * (GRP2 * CH), GRP2)
        return carry

    lax.fori_loop(0, ngrp2, body, 0)
    for t in range(NCHUNK - ngrp2 * GRP2):
        group(base + (ngrp2 * GRP2 + t) * CH, 1)


def _sc_gather_spmem(table, idx):
    return pl.kernel(
        _gather_spmem_body,
        out_type=jax.ShapeDtypeStruct((E, D), jnp.float32),
        mesh=_MESH,
        scratch_types=(
            [pltpu.VMEM_SHARED((NPAD, D), jnp.float32)]
            + [pltpu.VMEM((CH,), jnp.int32) for _ in range(GRP2)]
            + [pltpu.VMEM((CH, D), jnp.float32) for _ in range(GRP2)]
            + [pltpu.SemaphoreType.DMA] * 4
        ),
    )(table, idx)


CHS = 40
NCHUNKS = EW // CHS
SGRP = 8
SNGRP = NCHUNKS // SGRP
STAIL = NCHUNKS - SNGRP * SGRP
ZCP = RPT // CHS


def _scatter_body(vals, idx_hbm, out, acc, *scratch):
    idx_v = scratch[0:SGRP]
    rows_v = scratch[SGRP:2 * SGRP]
    isem, vsem, ssem, zsem = scratch[2 * SGRP:2 * SGRP + 4]
    c = lax.axis_index("c")
    s = lax.axis_index("s")
    wid = s * NC + c

    def zrow(r, carry):
        for k in range(D // 16):
            rows_v[0][r, pl.ds(k * 16, 16)] = jnp.zeros((16,), jnp.float32)
        return carry

    lax.fori_loop(0, CHS, zrow, 0)

    r0 = s * RPT
    zdesc = [
        pltpu.async_copy(rows_v[0], acc.at[pl.ds(r0 + k * CHS, CHS)], zsem)
        for k in range(ZCP)
    ]
    for d in zdesc:
        d.wait()
    plsc.subcore_barrier()

    base = wid * EW

    def body(i, carry):
        off0 = base + i * (SGRP * CHS)
        idesc = [
            pltpu.async_copy(idx_hbm.at[pl.ds(off0 + b * CHS, CHS)], idx_v[b], isem)
            for b in range(SGRP)
        ]
        vdesc = [
            pltpu.async_copy(vals.at[pl.ds(off0 + b * CHS, CHS)], rows_v[b], vsem)
            for b in range(SGRP)
        ]
        sdesc = []
        for b in range(SGRP):
            idesc[b].wait()
            vdesc[b].wait()
            sdesc.append(
                pltpu.async_copy(rows_v[b], acc.at[idx_v[b]], ssem, add=True)
            )
        for b in range(SGRP):
            sdesc[b].wait()
        return carry

    lax.fori_loop(0, SNGRP, body, 0)

    for t in range(STAIL):
        off = base + (SNGRP * SGRP + t) * CHS
        pltpu.sync_copy(idx_hbm.at[pl.ds(off, CHS)], idx_v[0])
        pltpu.sync_copy(vals.at[pl.ds(off, CHS)], rows_v[0])
        pltpu.sync_copy(rows_v[0], acc.at[idx_v[0]], add=True)

    plsc.subcore_barrier()

    wdesc = [
        pltpu.async_copy(
            acc.at[pl.ds(r0 + k * RCH, RCH)],
            out.at[pl.ds(c * NPAD + r0 + k * RCH, RCH)],
            zsem,
        )
        for k in range(NRCH)
    ]
    for d in wdesc:
        d.wait()


def _sc_scatter(vals, idx):
    return pl.kernel(
        _scatter_body,
        out_type=jax.ShapeDtypeStruct((2 * NPAD, D), jnp.float32),
        mesh=_MESH,
        scratch_types=(
            [pltpu.VMEM_SHARED((NPAD, D), jnp.float32)]
            + [pltpu.VMEM((CHS,), jnp.int32) for _ in range(SGRP)]
            + [pltpu.VMEM((CHS, D), jnp.float32) for _ in range(SGRP)]
            + [pltpu.SemaphoreType.DMA] * 4
        ),
    )(vals, idx)


NB = 1000
NGRID = N // NB
EB = 10000
EGRID = E // EB
AB = 1024
AGRID = NPAD // AB


def _xw_body(x_ref, w_ref, b_ref, o_ref):
    o_ref[...] = (
        jnp.dot(x_ref[...], w_ref[...], preferred_element_type=jnp.float32)
        + b_ref[...]
    )


def _tc_node_xw(x, w, b):
    return pl.pallas_call(
        _xw_body,
        grid=(NGRID,),
        in_specs=[
            pl.BlockSpec((NB, D), lambda i: (i, 0)),
            pl.BlockSpec((D, D), lambda i: (0, 0)),
            pl.BlockSpec((1, D), lambda i: (0, 0)),
        ],
        out_specs=pl.BlockSpec((NB, D), lambda i: (i, 0)),
        out_shape=jax.ShapeDtypeStruct((N, D), jnp.float32),
    )(x, w, b)


def _h0_body(gx_ref, ef_ref, w_ref, oh_ref, o16_ref):
    h = jnp.maximum(
        gx_ref[...]
        + jnp.dot(ef_ref[...], w_ref[...], preferred_element_type=jnp.float32),
        0.0,
    )
    oh_ref[...] = h
    o16_ref[...] = h.astype(jnp.bfloat16)


def _tc_edge_h0(gx, ef, w):
    return pl.pallas_call(
        _h0_body,
        grid=(EGRID,),
        in_specs=[
            pl.BlockSpec((EB, D), lambda i: (i, 0)),
            pl.BlockSpec((EB, D_EDGE), lambda i: (i, 0)),
            pl.BlockSpec((D_EDGE, D), lambda i: (0, 0)),
        ],
        out_specs=[
            pl.BlockSpec((EB, D), lambda i: (i, 0)),
            pl.BlockSpec((EB, D), lambda i: (i, 0)),
        ],
        out_shape=[
            jax.ShapeDtypeStruct((E, D), jnp.float32),
            jax.ShapeDtypeStruct((E, D), jnp.bfloat16),
        ],
    )(gx, ef, w)


def _a_body(p0_ref, p1_ref, w_ref, b_ref, o_ref):
    o_ref[...] = (
        jnp.dot(
            p0_ref[...] + p1_ref[...], w_ref[...],
            preferred_element_type=jnp.float32,
        )
        + b_ref[...]
    )


def _tc_node_a(p, w, b):
    return pl.pallas_call(
        _a_body,
        grid=(AGRID,),
        in_specs=[
            pl.BlockSpec((AB, D), lambda i: (i, 0)),
            pl.BlockSpec((AB, D), lambda i: (i + AGRID, 0)),
            pl.BlockSpec((D, D), lambda i: (0, 0)),
            pl.BlockSpec((1, D), lambda i: (0, 0)),
        ],
        out_specs=pl.BlockSpec((AB, D), lambda i: (i, 0)),
        out_shape=jax.ShapeDtypeStruct((NPAD, D), jnp.float32),
    )(p, p, w, b)


def _step_body(g_ref, h_ref, h0_ref, w1_ref, w2_ref, b2_ref, o_ref):
    t = jnp.maximum(
        g_ref[...]
        - jnp.dot(h_ref[...], w1_ref[...], preferred_element_type=jnp.float32),
        0.0,
    )
    o_ref[...] = jnp.maximum(
        h0_ref[...].astype(jnp.float32)
        + jnp.dot(t, w2_ref[...], preferred_element_type=jnp.float32)
        + b2_ref[...],
        0.0,
    )


def _tc_edge_step(g, h, h0, w1, w2, b2):
    return pl.pallas_call(
        _step_body,
        grid=(EGRID,),
        in_specs=[
            pl.BlockSpec((EB, D), lambda i: (i, 0)),
            pl.BlockSpec((EB, D), lambda i: (i, 0)),
            pl.BlockSpec((EB, D), lambda i: (i, 0)),
            pl.BlockSpec((D, D), lambda i: (0, 0)),
            pl.BlockSpec((D, D), lambda i: (0, 0)),
            pl.BlockSpec((1, D), lambda i: (0, 0)),
        ],
        out_specs=pl.BlockSpec((EB, D), lambda i: (i, 0)),
        out_shape=jax.ShapeDtypeStruct((E, D), jnp.float32),
    )(g, h, h0, w1, w2, b2)


def _final_body(x_ref, p0_ref, p1_ref, wx_ref, wm_ref, b_ref, g_ref, be_ref, o_ref):
    hf = jnp.maximum(
        jnp.dot(x_ref[...], wx_ref[...], preferred_element_type=jnp.float32)
        + jnp.dot(
            p0_ref[...] + p1_ref[...], wm_ref[...],
            preferred_element_type=jnp.float32,
        )
        + b_ref[...],
        0.0,
    )
    mu = jnp.mean(hf, axis=1, keepdims=True)
    d = hf - mu
    var = jnp.mean(d * d, axis=1, keepdims=True)
    hn = d * lax.rsqrt(var + 1e-5) * g_ref[...] + be_ref[...]
    o_ref[...] = jnp.maximum(hn, 0.0)


def _tc_node_final(x, p0, p1, wx, wm, b, gam, bet):
    return pl.pallas_call(
        _final_body,
        grid=(NGRID,),
        in_specs=[
            pl.BlockSpec((NB, D), lambda i: (i, 0)),
            pl.BlockSpec((NB, D), lambda i: (i, 0)),
            pl.BlockSpec((NB, D), lambda i: (i, 0)),
            pl.BlockSpec((D, D), lambda i: (0, 0)),
            pl.BlockSpec((D, D), lambda i: (0, 0)),
            pl.BlockSpec((1, D), lambda i: (0, 0)),
            pl.BlockSpec((1, D), lambda i: (0, 0)),
            pl.BlockSpec((1, D), lambda i: (0, 0)),
        ],
        out_specs=pl.BlockSpec((NB, D), lambda i: (i, 0)),
        out_shape=jax.ShapeDtypeStruct((N, D), jnp.float32),
    )(x, p0, p1, wx, wm, b, gam, bet)


def kernel(x, edge_feats, edge_index, W_init, b_init, W_h1, b_h1, W_h2, b_h2,
           W_final, b_final, ln_gamma, ln_beta):
    src = edge_index[0]
    dst = edge_index[1]
    Wi1 = W_init[:D]
    Wi2 = W_init[D:]
    Wf1 = W_final[:D]
    Wf2 = W_final[D:]
    bi = b_init.reshape(1, D)
    b1 = b_h1.reshape(1, D)
    b2 = b_h2.reshape(1, D)
    bf = b_final.reshape(1, D)
    gam = ln_gamma.reshape(1, D)
    bet = ln_beta.reshape(1, D)

    XW = _tc_node_xw(x, Wi1, bi)
    GX = _sc_gather(XW, src)
    h, h0 = _tc_edge_h0(GX, edge_feats, Wi2)
    for _ in range(STEPS):
        P = _sc_scatter(h, dst)
        A = _tc_node_a(P, W_h1, b1)
        G = _sc_gather_spmem(A, dst)
        h = _tc_edge_step(G, h, h0, W_h1, W_h2, b2)
    P = _sc_scatter(h, src)
    return _tc_node_final(x, P[:N], P[NPAD:NPAD + N], Wf1, Wf2, bf, gam, bet)


# speedup vs baseline: 1.2659x; 1.0380x over previous
import jax
import jax.numpy as jnp
from jax import lax
from jax.experimental import pallas as pl
from jax.experimental.pallas import tpu as pltpu
from jax.experimental.pallas import tpu_sc as plsc

N = 10000
E = 320000
D = 128
D_EDGE = 16
STEPS = 3

NC, NS = 2, 16
NW = NC * NS
EW = E // NW
CH = 80
NCHUNK = EW // CH
NPAD = 10240
RPT = NPAD // NS
RCH = 128
NRCH = RPT // RCH

_MESH = plsc.VectorSubcoreMesh(core_axis_name="c", subcore_axis_name="s")


GRP = 8
NGRP = NCHUNK // GRP
GTAIL = NCHUNK - NGRP * GRP


def _gather_body(table, idx_hbm, out, *scratch):
    idx_v = scratch[0:GRP]
    rows_v = scratch[GRP:2 * GRP]
    isem, gsem, wsem = scratch[2 * GRP:2 * GRP + 3]
    c = lax.axis_index("c")
    s = lax.axis_index("s")
    wid = s * NC + c
    base = wid * EW

    def group(off0, n):
        idesc = [
            pltpu.async_copy(idx_hbm.at[pl.ds(off0 + b * CH, CH)], idx_v[b], isem)
            for b in range(n)
        ]
        gdesc = []
        for b in range(n):
            idesc[b].wait()
            gdesc.append(pltpu.async_copy(table.at[idx_v[b]], rows_v[b], gsem))
        wdesc = []
        for b in range(n):
            gdesc[b].wait()
            wdesc.append(
                pltpu.async_copy(rows_v[b], out.at[pl.ds(off0 + b * CH, CH)], wsem)
            )
        for b in range(n):
            wdesc[b].wait()

    def body(i, carry):
        group(base + i * (GRP * CH), GRP)
        return carry

    lax.fori_loop(0, NGRP, body, 0)
    if GTAIL:
        group(base + NGRP * GRP * CH, GTAIL)


def _sc_gather(table, idx):
    width = table.shape[1]
    dtype = table.dtype
    return pl.kernel(
        _gather_body,
        out_type=jax.ShapeDtypeStruct((E, width), dtype),
        mesh=_MESH,
        scratch_types=(
            [pltpu.VMEM((CH,), jnp.int32) for _ in range(GRP)]
            + [pltpu.VMEM((CH, width), dtype) for _ in range(GRP)]
            + [pltpu.SemaphoreType.DMA] * 3
        ),
    )(table, idx)


GRP2 = 4


def _gather_spmem_body(table, idx_hbm, out, tbl, *scratch):
    idx_v = scratch[0:GRP2]
    rows_v = scratch[GRP2:2 * GRP2]
    isem, gsem, wsem, lsem = scratch[2 * GRP2:2 * GRP2 + 4]
    c = lax.axis_index("c")
    s = lax.axis_index("s")
    wid = s * NC + c
    base = wid * EW

    r0 = s * RPT
    ldesc = [
        pltpu.async_copy(
            table.at[pl.ds(r0 + k * RCH, RCH)], tbl.at[pl.ds(r0 + k * RCH, RCH)],
            lsem,
        )
        for k in range(NRCH)
    ]
    for d in ldesc:
        d.wait()
    plsc.subcore_barrier()

    def group(off0, n):
        idesc = [
            pltpu.async_copy(idx_hbm.at[pl.ds(off0 + b * CH, CH)], idx_v[b], isem)
            for b in range(n)
        ]
        gdesc = []
        for b in range(n):
            idesc[b].wait()
            gdesc.append(pltpu.async_copy(tbl.at[idx_v[b]], rows_v[b], gsem))
        wdesc = []
        for b in range(n):
            gdesc[b].wait()
            wdesc.append(
                pltpu.async_copy(rows_v[b], out.at[pl.ds(off0 + b * CH, CH)], wsem)
            )
        for b in range(n):
            wdesc[b].wait()

    ngrp2 = NCHUNK // GRP2
    def body(i, carry):
        group(base + i * (GRP2 * CH), GRP2)
        return carry

    lax.fori_loop(0, ngrp2, body, 0)
    for t in range(NCHUNK - ngrp2 * GRP2):
        group(base + (ngrp2 * GRP2 + t) * CH, 1)


def _sc_gather_spmem(table, idx):
    return pl.kernel(
        _gather_spmem_body,
        out_type=jax.ShapeDtypeStruct((E, D), jnp.float32),
        mesh=_MESH,
        scratch_types=(
            [pltpu.VMEM_SHARED((NPAD, D), jnp.float32)]
            + [pltpu.VMEM((CH,), jnp.int32) for _ in range(GRP2)]
            + [pltpu.VMEM((CH, D), jnp.float32) for _ in range(GRP2)]
            + [pltpu.SemaphoreType.DMA] * 4
        ),
    )(table, idx)


CHS = 40
NCHUNKS = EW // CHS
SGRP = 8
SNGRP = NCHUNKS // SGRP
STAIL = NCHUNKS - SNGRP * SGRP
ZCP = RPT // CHS


def _scatter_body(vals, idx_hbm, out, acc, *scratch):
    idx_v = scratch[0:SGRP]
    rows_v = scratch[SGRP:2 * SGRP]
    isem, vsem, ssem, zsem = scratch[2 * SGRP:2 * SGRP + 4]
    c = lax.axis_index("c")
    s = lax.axis_index("s")
    wid = s * NC + c

    def zrow(r, carry):
        for k in range(D // 16):
            rows_v[0][r, pl.ds(k * 16, 16)] = jnp.zeros((16,), jnp.float32)
        return carry

    lax.fori_loop(0, CHS, zrow, 0)

    r0 = s * RPT
    zdesc = [
        pltpu.async_copy(rows_v[0], acc.at[pl.ds(r0 + k * CHS, CHS)], zsem)
        for k in range(ZCP)
    ]
    for d in zdesc:
        d.wait()
    plsc.subcore_barrier()

    base = wid * EW

    def body(i, carry):
        off0 = base + i * (SGRP * CHS)
        idesc = [
            pltpu.async_copy(idx_hbm.at[pl.ds(off0 + b * CHS, CHS)], idx_v[b], isem)
            for b in range(SGRP)
        ]
        vdesc = [
            pltpu.async_copy(vals.at[pl.ds(off0 + b * CHS, CHS)], rows_v[b], vsem)
            for b in range(SGRP)
        ]
        sdesc = []
        for b in range(SGRP):
            idesc[b].wait()
            vdesc[b].wait()
            sdesc.append(
                pltpu.async_copy(rows_v[b], acc.at[idx_v[b]], ssem, add=True)
            )
        for b in range(SGRP):
            sdesc[b].wait()
        return carry

    lax.fori_loop(0, SNGRP, body, 0)

    for t in range(STAIL):
        off = base + (SNGRP * SGRP + t) * CHS
        pltpu.sync_copy(idx_hbm.at[pl.ds(off, CHS)], idx_v[0])
        pltpu.sync_copy(vals.at[pl.ds(off, CHS)], rows_v[0])
        pltpu.sync_copy(rows_v[0], acc.at[idx_v[0]], add=True)

    plsc.subcore_barrier()

    wdesc = [
        pltpu.async_copy(
            acc.at[pl.ds(r0 + k * RCH, RCH)],
            out.at[pl.ds(c * NPAD + r0 + k * RCH, RCH)],
            zsem,
        )
        for k in range(NRCH)
    ]
    for d in wdesc:
        d.wait()


def _sc_scatter(vals, idx):
    return pl.kernel(
        _scatter_body,
        out_type=jax.ShapeDtypeStruct((2 * NPAD, D), jnp.float32),
        mesh=_MESH,
        scratch_types=(
            [pltpu.VMEM_SHARED((NPAD, D), jnp.float32)]
            + [pltpu.VMEM((CHS,), jnp.int32) for _ in range(SGRP)]
            + [pltpu.VMEM((CHS, D), jnp.float32) for _ in range(SGRP)]
            + [pltpu.SemaphoreType.DMA] * 4
        ),
    )(vals, idx)


NB = 1000
NGRID = N // NB
EB = 10000
EGRID = E // EB
AB = 1024
AGRID = NPAD // AB


def _xw_body(x_ref, w_ref, b_ref, o_ref):
    o_ref[...] = (
        jnp.dot(x_ref[...], w_ref[...], preferred_element_type=jnp.float32)
        + b_ref[...]
    )


def _tc_node_xw(x, w, b):
    return pl.pallas_call(
        _xw_body,
        grid=(AGRID,),
        in_specs=[
            pl.BlockSpec((AB, D), lambda i: (i, 0)),
            pl.BlockSpec((D, D), lambda i: (0, 0)),
            pl.BlockSpec((1, D), lambda i: (0, 0)),
        ],
        out_specs=pl.BlockSpec((AB, D), lambda i: (i, 0)),
        out_shape=jax.ShapeDtypeStruct((NPAD, D), jnp.float32),
    )(x, w, b)


def _h0_body(gx_ref, ef_ref, w_ref, oh_ref, o16_ref):
    h = jnp.maximum(
        gx_ref[...]
        + jnp.dot(ef_ref[...], w_ref[...], preferred_element_type=jnp.float32),
        0.0,
    )
    oh_ref[...] = h
    o16_ref[...] = h.astype(jnp.bfloat16)


def _tc_edge_h0(gx, ef, w):
    return pl.pallas_call(
        _h0_body,
        grid=(EGRID,),
        in_specs=[
            pl.BlockSpec((EB, D), lambda i: (i, 0)),
            pl.BlockSpec((EB, D_EDGE), lambda i: (i, 0)),
            pl.BlockSpec((D_EDGE, D), lambda i: (0, 0)),
        ],
        out_specs=[
            pl.BlockSpec((EB, D), lambda i: (i, 0)),
            pl.BlockSpec((EB, D), lambda i: (i, 0)),
        ],
        out_shape=[
            jax.ShapeDtypeStruct((E, D), jnp.float32),
            jax.ShapeDtypeStruct((E, D), jnp.bfloat16),
        ],
    )(gx, ef, w)


def _a_body(p0_ref, p1_ref, w_ref, b_ref, o_ref):
    o_ref[...] = (
        jnp.dot(
            p0_ref[...] + p1_ref[...], w_ref[...],
            preferred_element_type=jnp.float32,
        )
        + b_ref[...]
    )


def _tc_node_a(p, w, b):
    return pl.pallas_call(
        _a_body,
        grid=(AGRID,),
        in_specs=[
            pl.BlockSpec((AB, D), lambda i: (i, 0)),
            pl.BlockSpec((AB, D), lambda i: (i + AGRID, 0)),
            pl.BlockSpec((D, D), lambda i: (0, 0)),
            pl.BlockSpec((1, D), lambda i: (0, 0)),
        ],
        out_specs=pl.BlockSpec((AB, D), lambda i: (i, 0)),
        out_shape=jax.ShapeDtypeStruct((NPAD, D), jnp.float32),
    )(p, p, w, b)


def _step_body(g_ref, h_ref, h0_ref, w1_ref, w2_ref, b2_ref, o_ref):
    t = jnp.maximum(
        g_ref[...]
        - jnp.dot(h_ref[...], w1_ref[...], preferred_element_type=jnp.float32),
        0.0,
    )
    o_ref[...] = jnp.maximum(
        h0_ref[...].astype(jnp.float32)
        + jnp.dot(t, w2_ref[...], preferred_element_type=jnp.float32)
        + b2_ref[...],
        0.0,
    )


def _tc_edge_step(g, h, h0, w1, w2, b2):
    return pl.pallas_call(
        _step_body,
        grid=(EGRID,),
        in_specs=[
            pl.BlockSpec((EB, D), lambda i: (i, 0)),
            pl.BlockSpec((EB, D), lambda i: (i, 0)),
            pl.BlockSpec((EB, D), lambda i: (i, 0)),
            pl.BlockSpec((D, D), lambda i: (0, 0)),
            pl.BlockSpec((D, D), lambda i: (0, 0)),
            pl.BlockSpec((1, D), lambda i: (0, 0)),
        ],
        out_specs=pl.BlockSpec((EB, D), lambda i: (i, 0)),
        out_shape=jax.ShapeDtypeStruct((E, D), jnp.float32),
    )(g, h, h0, w1, w2, b2)


def _final_body(x_ref, p0_ref, p1_ref, wx_ref, wm_ref, b_ref, g_ref, be_ref, o_ref):
    hf = jnp.maximum(
        jnp.dot(x_ref[...], wx_ref[...], preferred_element_type=jnp.float32)
        + jnp.dot(
            p0_ref[...] + p1_ref[...], wm_ref[...],
            preferred_element_type=jnp.float32,
        )
        + b_ref[...],
        0.0,
    )
    mu = jnp.mean(hf, axis=1, keepdims=True)
    d = hf - mu
    var = jnp.mean(d * d, axis=1, keepdims=True)
    hn = d * lax.rsqrt(var + 1e-5) * g_ref[...] + be_ref[...]
    o_ref[...] = jnp.maximum(hn, 0.0)


def _tc_node_final(x, p0, p1, wx, wm, b, gam, bet):
    return pl.pallas_call(
        _final_body,
        grid=(NGRID,),
        in_specs=[
            pl.BlockSpec((NB, D), lambda i: (i, 0)),
            pl.BlockSpec((NB, D), lambda i: (i, 0)),
            pl.BlockSpec((NB, D), lambda i: (i, 0)),
            pl.BlockSpec((D, D), lambda i: (0, 0)),
            pl.BlockSpec((D, D), lambda i: (0, 0)),
            pl.BlockSpec((1, D), lambda i: (0, 0)),
            pl.BlockSpec((1, D), lambda i: (0, 0)),
            pl.BlockSpec((1, D), lambda i: (0, 0)),
        ],
        out_specs=pl.BlockSpec((NB, D), lambda i: (i, 0)),
        out_shape=jax.ShapeDtypeStruct((N, D), jnp.float32),
    )(x, p0, p1, wx, wm, b, gam, bet)


def kernel(x, edge_feats, edge_index, W_init, b_init, W_h1, b_h1, W_h2, b_h2,
           W_final, b_final, ln_gamma, ln_beta):
    src = edge_index[0]
    dst = edge_index[1]
    Wi1 = W_init[:D]
    Wi2 = W_init[D:]
    Wf1 = W_final[:D]
    Wf2 = W_final[D:]
    bi = b_init.reshape(1, D)
    b1 = b_h1.reshape(1, D)
    b2 = b_h2.reshape(1, D)
    bf = b_final.reshape(1, D)
    gam = ln_gamma.reshape(1, D)
    bet = ln_beta.reshape(1, D)

    XW = _tc_node_xw(x, Wi1, bi)
    GX = _sc_gather_spmem(XW, src)
    h, h0 = _tc_edge_h0(GX, edge_feats, Wi2)
    for _ in range(STEPS):
        P = _sc_scatter(h, dst)
        A = _tc_node_a(P, W_h1, b1)
        G = _sc_gather_spmem(A, dst)
        h = _tc_edge_step(G, h, h0, W_h1, W_h2, b2)
    P = _sc_scatter(h, src)
    return _tc_node_final(x, P[:N], P[NPAD:NPAD + N], Wf1, Wf2, bf, gam, bet)
